# Initial kernel scaffold; baseline (speedup 1.0000x reference)
import jax
import jax.numpy as jnp
from jax.experimental import pallas as pl


def kernel(inputs):
    raise NotImplementedError("write your pallas kernel here")


# speedup vs baseline: 20.1233x; 20.1233x over previous
import jax
import jax.numpy as jnp
from jax.experimental import pallas as pl

def _median_body(x_ref, o_ref, *, n, rows):
    nc = 2 * n
    half = n // 2
    _I32_MIN = jnp.int32(-(2**31))
    _I32_MAX = jnp.int32(2**31 - 1)
    _SIGNMASK = jnp.int32(0x7FFFFFFF)
    x = x_ref[...].reshape(rows, nc)
    b = jax.lax.bitcast_convert_type(x, jnp.int32)
    s = jnp.where(b < 0, b ^ _SIGNMASK, b)
    lane = jax.lax.broadcasted_iota(jnp.int32, (rows, nc), 1)
    evi = jnp.where(lane % 2 == 0, jnp.int32(1), jnp.int32(0))
    ev = lane % 2 == 0

    def bcast2(v):
        c0 = jnp.broadcast_to(v[:, 0:1], (rows, nc))
        c1 = jnp.broadcast_to(v[:, 1:2], (rows, nc))
        return jnp.where(ev, c0, c1)

    def parity_counts(ci):
        tot = jnp.sum(ci, axis=1, keepdims=True)
        c0 = jnp.sum(ci * evi, axis=1, keepdims=True)
        return jnp.concatenate([c0, tot - c0], axis=1)

    def step(i, p_u):
        bit = jax.lax.shift_left(jnp.int32(1), 31 - i)
        cand_u = p_u | bit
        cand_s = cand_u ^ _I32_MIN
        cmp = s < bcast2(cand_s)
        cnts = parity_counts(jnp.where(cmp, jnp.int32(1), jnp.int32(0)))
        take = cnts <= jnp.int32(half - 1)
        return jnp.where(take, cand_u, p_u)

    p_u = jax.lax.fori_loop(0, 32, step, jnp.zeros((rows, 2), jnp.int32))
    s_lo = p_u ^ _I32_MIN

    le = s <= bcast2(s_lo)
    c_le = parity_counts(jnp.where(le, jnp.int32(1), jnp.int32(0)))
    s_ab = jnp.where(le, _I32_MAX, s)
    m0 = jnp.min(jnp.where(ev, s_ab, _I32_MAX), axis=1, keepdims=True)
    m1 = jnp.min(jnp.where(ev, _I32_MAX, s_ab), axis=1, keepdims=True)
    m_above = jnp.concatenate([m0, m1], axis=1)
    s_hi = jnp.where(c_le >= jnp.int32(half + 1), s_lo, m_above)

    def to_f32(sk):
        return jax.lax.bitcast_convert_type(
            jnp.where(sk < 0, sk ^ _SIGNMASK, sk), jnp.float32)

    o_ref[...] = (to_f32(s_lo) + to_f32(s_hi)) * jnp.float32(0.5)


def kernel(inputs):
    b, n, c = inputs.shape
    assert c == 2 and n % 2 == 0
    rows = 128 if b % 128 == 0 else 8
    x2d = inputs.reshape(b, n * c)
    import functools
    body = functools.partial(_median_body, n=n, rows=rows)
    out = pl.pallas_call(
        body,
        grid=(b // rows,),
        in_specs=[pl.BlockSpec((rows, n * c), lambda i: (i, 0))],
        out_specs=pl.BlockSpec((rows, c), lambda i: (i, 0)),
        out_shape=jax.ShapeDtypeStruct((b, c), jnp.float32),
    )(x2d)
    return out.reshape(b, 1, c)


# speedup vs baseline: 26.0815x; 1.2961x over previous
import jax
import jax.numpy as jnp
from jax.experimental import pallas as pl

def _median_body(x_ref, o_ref, *, n, rows):
    nc = 2 * n
    half = n // 2
    _I32_MIN = jnp.int32(-(2**31))
    _I32_MAX = jnp.int32(2**31 - 1)
    _SIGNMASK = jnp.int32(0x7FFFFFFF)
    x = x_ref[...].reshape(rows, nc)
    b = jax.lax.bitcast_convert_type(x, jnp.int32)
    s = jnp.where(b < 0, b ^ _SIGNMASK, b)
    lane = jax.lax.broadcasted_iota(jnp.int32, (rows, nc), 1)
    evi = jnp.where(lane % 2 == 0, jnp.int32(1), jnp.int32(0))
    ev = lane % 2 == 0

    def bcast2(v):
        c0 = jnp.broadcast_to(v[:, 0:1], (rows, nc))
        c1 = jnp.broadcast_to(v[:, 1:2], (rows, nc))
        return jnp.where(ev, c0, c1)

    payload = jnp.where(ev, jnp.int32(1), jnp.int32(1 << 16))

    def parity_counts(cmp):
        packed = jnp.sum(jnp.where(cmp, payload, jnp.int32(0)),
                         axis=1, keepdims=True)
        c0 = packed & jnp.int32(0xFFFF)
        c1 = jax.lax.shift_right_logical(packed, 16)
        return jnp.concatenate([c0, c1], axis=1)

    def step(i, p_u):
        bit = jax.lax.shift_left(jnp.int32(1), 31 - i)
        cand_u = p_u | bit
        cand_s = cand_u ^ _I32_MIN
        cnts = parity_counts(s < bcast2(cand_s))
        take = cnts <= jnp.int32(half - 1)
        return jnp.where(take, cand_u, p_u)

    p_u = jax.lax.fori_loop(0, 32, step, jnp.zeros((rows, 2), jnp.int32))
    s_lo = p_u ^ _I32_MIN

    le = s <= bcast2(s_lo)
    c_le = parity_counts(le)
    s_ab = jnp.where(le, _I32_MAX, s)
    m0 = jnp.min(jnp.where(ev, s_ab, _I32_MAX), axis=1, keepdims=True)
    m1 = jnp.min(jnp.where(ev, _I32_MAX, s_ab), axis=1, keepdims=True)
    m_above = jnp.concatenate([m0, m1], axis=1)
    s_hi = jnp.where(c_le >= jnp.int32(half + 1), s_lo, m_above)

    def to_f32(sk):
        return jax.lax.bitcast_convert_type(
            jnp.where(sk < 0, sk ^ _SIGNMASK, sk), jnp.float32)

    o_ref[...] = (to_f32(s_lo) + to_f32(s_hi)) * jnp.float32(0.5)


def kernel(inputs):
    b, n, c = inputs.shape
    assert c == 2 and n % 2 == 0
    rows = 128 if b % 128 == 0 else 8
    x2d = inputs.reshape(b, n * c)
    import functools
    body = functools.partial(_median_body, n=n, rows=rows)
    out = pl.pallas_call(
        body,
        grid=(b // rows,),
        in_specs=[pl.BlockSpec((rows, n * c), lambda i: (i, 0))],
        out_specs=pl.BlockSpec((rows, c), lambda i: (i, 0)),
        out_shape=jax.ShapeDtypeStruct((b, c), jnp.float32),
    )(x2d)
    return out.reshape(b, 1, c)
